# Initial kernel scaffold; baseline (speedup 1.0000x reference)
import jax
import jax.numpy as jnp
from jax.experimental import pallas as pl


def kernel(month, hour, query, last_viewed, month_table, hour_table, query_table, lastv_table, W1, b1, W2, b2, W3, b3):
    raise NotImplementedError("write your pallas kernel here")


# speedup vs baseline: 13.2720x; 13.2720x over previous
import functools

import jax
import jax.numpy as jnp
from jax import lax
from jax.experimental import pallas as pl
from jax.experimental.pallas import tpu as pltpu
from jax.experimental.pallas import tpu_sc as plsc

_B = 16384
_D = 32
_L = 50
_NW = 32
_ROWS_W = _B // _NW
_CH = 8
_IDXROWS = _CH // 2
_NCHUNK = _ROWS_W // _CH


def _sc_pool_sums(q_idx2, lv_idx2, q_table, lv_table):
    mesh = plsc.VectorSubcoreMesh(core_axis_name="c", subcore_axis_name="s")

    @functools.partial(
        pl.kernel,
        mesh=mesh,
        out_type=(
            jax.ShapeDtypeStruct((_B, _D), jnp.float32),
            jax.ShapeDtypeStruct((_B, _D), jnp.float32),
        ),
        scratch_types=[
            pltpu.VMEM((_IDXROWS, 2 * _L), jnp.int32),
            pltpu.VMEM((_IDXROWS, 2 * _L, _D), jnp.float32),
            pltpu.VMEM((_CH, _D), jnp.float32),
            pltpu.SemaphoreType.DMA,
        ],
        compiler_params=pltpu.CompilerParams(use_tc_tiling_on_sc=False),
    )
    def k(qi, lvi, qt, lvt, out_q, out_lv, idx_v, gat_v, sums_v, sem):
        cid = lax.axis_index("c")
        sid = lax.axis_index("s")
        wid = sid * 2 + cid

        for idx_hbm, tab_hbm, out_hbm in ((qi, qt, out_q), (lvi, lvt, out_lv)):

            def chunk_body(c, _, idx_hbm=idx_hbm, tab_hbm=tab_hbm,
                           out_hbm=out_hbm):
                base = wid * _ROWS_W + c * _CH
                ib = wid * (_ROWS_W // 2) + c * _IDXROWS
                pltpu.sync_copy(idx_hbm.at[pl.ds(ib, _IDXROWS)], idx_v)
                copies = [
                    pltpu.async_copy(tab_hbm.at[idx_v.at[j]], gat_v.at[j], sem)
                    for j in range(_IDXROWS)
                ]
                for cp in copies:
                    cp.wait()
                for r in range(_CH):
                    j = r // 2
                    o = (r % 2) * _L
                    a = [gat_v[j, o + t, pl.ds(0, 16)] for t in range(4)]
                    b = [gat_v[j, o + t, pl.ds(16, 16)] for t in range(4)]
                    for t in range(4, _L):
                        a[t % 4] = a[t % 4] + gat_v[j, o + t, pl.ds(0, 16)]
                        b[t % 4] = b[t % 4] + gat_v[j, o + t, pl.ds(16, 16)]
                    sums_v[r, pl.ds(0, 16)] = (a[0] + a[1]) + (a[2] + a[3])
                    sums_v[r, pl.ds(16, 16)] = (b[0] + b[1]) + (b[2] + b[3])
                pltpu.sync_copy(sums_v, out_hbm.at[pl.ds(base, _CH)])
                return _

            lax.fori_loop(0, _NCHUNK, chunk_body, 0)

    return k(q_idx2, lv_idx2, q_table, lv_table)


def _tc_head(month, hour, query, lastv, mt16, ht32, r0, sum_q, sum_lv,
             W1, b1, W2, b2, W3, b3):
    bm = 2048
    grid = _B // bm
    hp = lax.Precision.HIGHEST
    f32 = jnp.float32

    def body(mo, ho, qu, lv, mt, ht, r0_, sq, slv,
             W1_, b1_, W2_, b2_, W3_, b3_, out):
        def pooled_small(idx, tab, ncls, npos):
            cnt = jnp.zeros((bm, ncls), f32)
            iot = lax.broadcasted_iota(jnp.int32, (bm, ncls), 1)
            for l in range(npos):
                cnt = cnt + (idx[:, l][:, None] == iot).astype(f32)
            full = jnp.dot(cnt, tab, precision=hp)
            n0 = cnt[:, 0:1]
            s = full - n0 * tab[0:1, :]
            return s / jnp.maximum(float(npos) - n0, 1.0)

        mo_, ho_, qu_, lv_ = mo[...], ho[...], qu[...], lv[...]
        m = pooled_small(mo_, mt[...], 16, 20)
        h = pooled_small(ho_, ht[...], 32, 20)

        n0q = jnp.sum((qu_ == 0).astype(f32), axis=1, keepdims=True)
        q = (sq[...] - n0q * r0_[0:1, :]) / jnp.maximum(50.0 - n0q, 1.0)
        n0l = jnp.sum((lv_ == 0).astype(f32), axis=1, keepdims=True)
        lvp = (slv[...] - n0l * r0_[1:2, :]) / jnp.maximum(50.0 - n0l, 1.0)

        x = jnp.concatenate([m, h, q, lvp], axis=1)
        x = jnp.maximum(jnp.dot(x, W1_[...], precision=hp) + b1_[...], 0.0)
        x = jnp.maximum(jnp.dot(x, W2_[...], precision=hp) + b2_[...], 0.0)
        out[...] = jnp.dot(x, W3_[...], precision=hp) + b3_[...]

    row_spec = lambda cols: pl.BlockSpec((bm, cols), lambda i: (i, 0))
    full_spec = lambda rows, cols: pl.BlockSpec((rows, cols), lambda i: (0, 0))
    return pl.pallas_call(
        body,
        grid=(grid,),
        in_specs=[
            row_spec(20), row_spec(20), row_spec(_L), row_spec(_L),
            full_spec(16, _D), full_spec(32, _D), full_spec(2, _D),
            row_spec(_D), row_spec(_D),
            full_spec(128, 128), full_spec(1, 128),
            full_spec(128, 64), full_spec(1, 64),
            full_spec(64, 32), full_spec(1, 32),
        ],
        out_specs=pl.BlockSpec((bm, _D), lambda i: (i, 0)),
        out_shape=jax.ShapeDtypeStruct((_B, _D), jnp.float32),
    )(month, hour, query, lastv, mt16, ht32, r0, sum_q, sum_lv,
      W1, b1, W2, b2, W3, b3)


def kernel(month, hour, query, last_viewed, month_table, hour_table,
           query_table, lastv_table, W1, b1, W2, b2, W3, b3):
    i32 = jnp.int32
    qu = query.astype(i32)
    lv = last_viewed.astype(i32)
    sum_q, sum_lv = _sc_pool_sums(
        qu.reshape(_B // 2, 2 * _L), lv.reshape(_B // 2, 2 * _L),
        query_table, lastv_table)
    mt16 = jnp.zeros((16, _D), jnp.float32).at[:13].set(month_table)
    ht32 = jnp.zeros((32, _D), jnp.float32).at[:26].set(hour_table)
    r0 = jnp.concatenate([query_table[0:1], lastv_table[0:1]], axis=0)
    return _tc_head(
        month.astype(i32), hour.astype(i32), qu, lv, mt16, ht32, r0,
        sum_q, sum_lv,
        W1, b1.reshape(1, -1), W2, b2.reshape(1, -1), W3, b3.reshape(1, -1))


# speedup vs baseline: 19.5236x; 1.4710x over previous
import functools

import jax
import jax.numpy as jnp
from jax import lax
from jax.experimental import pallas as pl
from jax.experimental.pallas import tpu as pltpu
from jax.experimental.pallas import tpu_sc as plsc

_B = 16384
_D = 32
_L = 50
_NW = 32
_ROWS_W = _B // _NW
_CH = 8
_IDXROWS = _CH // 2
_NCHUNK = _ROWS_W // _CH


_IR_W = _ROWS_W // 2
_NBUF = 4


def _sc_pool_sums(q_idx2, lv_idx2, q_table, lv_table):
    mesh = plsc.VectorSubcoreMesh(core_axis_name="c", subcore_axis_name="s")

    @functools.partial(
        pl.kernel,
        mesh=mesh,
        out_type=(
            jax.ShapeDtypeStruct((_B, _D), jnp.float32),
            jax.ShapeDtypeStruct((_B, _D), jnp.float32),
        ),
        scratch_types=[
            pltpu.VMEM((_IR_W, 2 * _L), jnp.int32),
            pltpu.VMEM((_NBUF, 2 * _L, _D), jnp.float32),
            pltpu.VMEM((_ROWS_W, _D), jnp.float32),
            [pltpu.SemaphoreType.DMA] * _NBUF,
        ],
        compiler_params=pltpu.CompilerParams(use_tc_tiling_on_sc=False),
    )
    def k(qi, lvi, qt, lvt, out_q, out_lv, idx_v, gat_v, sums_v, sems):
        cid = lax.axis_index("c")
        sid = lax.axis_index("s")
        wid = sid * 2 + cid

        def reduce_row(slot, o, dst):
            a = [gat_v[slot, o + t, pl.ds(0, 16)] for t in range(4)]
            b = [gat_v[slot, o + t, pl.ds(16, 16)] for t in range(4)]
            for t in range(4, _L):
                a[t % 4] = a[t % 4] + gat_v[slot, o + t, pl.ds(0, 16)]
                b[t % 4] = b[t % 4] + gat_v[slot, o + t, pl.ds(16, 16)]
            sums_v[dst, pl.ds(0, 16)] = (a[0] + a[1]) + (a[2] + a[3])
            sums_v[dst, pl.ds(16, 16)] = (b[0] + b[1]) + (b[2] + b[3])

        for idx_hbm, tab_hbm, out_hbm in ((qi, qt, out_q), (lvi, lvt, out_lv)):
            pltpu.sync_copy(idx_hbm.at[pl.ds(wid * _IR_W, _IR_W)], idx_v)
            for b in range(_NBUF):
                pltpu.async_copy(tab_hbm.at[idx_v.at[b]], gat_v.at[b],
                                 sems[b])

            def body(jj, carry, tab_hbm=tab_hbm):
                for b in range(_NBUF):
                    row = jj * _NBUF + b
                    pltpu.make_async_copy(
                        tab_hbm.at[idx_v.at[row]], gat_v.at[b],
                        sems[b]).wait()
                    reduce_row(b, 0, 2 * row)
                    reduce_row(b, _L, 2 * row + 1)

                    @pl.when(row + _NBUF < _IR_W)
                    def _refire(b=b, row=row):
                        pltpu.async_copy(
                            tab_hbm.at[idx_v.at[row + _NBUF]], gat_v.at[b],
                            sems[b])
                return carry

            lax.fori_loop(0, _IR_W // _NBUF, body, 0)
            pltpu.sync_copy(sums_v, out_hbm.at[pl.ds(wid * _ROWS_W, _ROWS_W)])

    return k(q_idx2, lv_idx2, q_table, lv_table)


def _tc_head(month, hour, query, lastv, mt16, ht32, r0, sum_q, sum_lv,
             W1, b1, W2, b2, W3, b3):
    bm = 2048
    grid = _B // bm
    hp = lax.Precision.HIGHEST
    f32 = jnp.float32

    def body(mo, ho, qu, lv, mt, ht, r0_, sq, slv,
             W1_, b1_, W2_, b2_, W3_, b3_, out):
        def pooled_small(idx, tab, ncls, npos):
            cnt = jnp.zeros((bm, ncls), f32)
            iot = lax.broadcasted_iota(jnp.int32, (bm, ncls), 1)
            for l in range(npos):
                cnt = cnt + (idx[:, l][:, None] == iot).astype(f32)
            full = jnp.dot(cnt, tab, precision=hp)
            n0 = cnt[:, 0:1]
            s = full - n0 * tab[0:1, :]
            return s / jnp.maximum(float(npos) - n0, 1.0)

        mo_, ho_, qu_, lv_ = mo[...], ho[...], qu[...], lv[...]
        m = pooled_small(mo_, mt[...], 16, 20)
        h = pooled_small(ho_, ht[...], 32, 20)

        n0q = jnp.sum((qu_ == 0).astype(f32), axis=1, keepdims=True)
        q = (sq[...] - n0q * r0_[0:1, :]) / jnp.maximum(50.0 - n0q, 1.0)
        n0l = jnp.sum((lv_ == 0).astype(f32), axis=1, keepdims=True)
        lvp = (slv[...] - n0l * r0_[1:2, :]) / jnp.maximum(50.0 - n0l, 1.0)

        x = jnp.concatenate([m, h, q, lvp], axis=1)
        x = jnp.maximum(jnp.dot(x, W1_[...], precision=hp) + b1_[...], 0.0)
        x = jnp.maximum(jnp.dot(x, W2_[...], precision=hp) + b2_[...], 0.0)
        out[...] = jnp.dot(x, W3_[...], precision=hp) + b3_[...]

    row_spec = lambda cols: pl.BlockSpec((bm, cols), lambda i: (i, 0))
    full_spec = lambda rows, cols: pl.BlockSpec((rows, cols), lambda i: (0, 0))
    return pl.pallas_call(
        body,
        grid=(grid,),
        in_specs=[
            row_spec(20), row_spec(20), row_spec(_L), row_spec(_L),
            full_spec(16, _D), full_spec(32, _D), full_spec(2, _D),
            row_spec(_D), row_spec(_D),
            full_spec(128, 128), full_spec(1, 128),
            full_spec(128, 64), full_spec(1, 64),
            full_spec(64, 32), full_spec(1, 32),
        ],
        out_specs=pl.BlockSpec((bm, _D), lambda i: (i, 0)),
        out_shape=jax.ShapeDtypeStruct((_B, _D), jnp.float32),
    )(month, hour, query, lastv, mt16, ht32, r0, sum_q, sum_lv,
      W1, b1, W2, b2, W3, b3)


def kernel(month, hour, query, last_viewed, month_table, hour_table,
           query_table, lastv_table, W1, b1, W2, b2, W3, b3):
    i32 = jnp.int32
    qu = query.astype(i32)
    lv = last_viewed.astype(i32)
    sum_q, sum_lv = _sc_pool_sums(
        qu.reshape(_B // 2, 2 * _L), lv.reshape(_B // 2, 2 * _L),
        query_table, lastv_table)
    mt16 = jnp.zeros((16, _D), jnp.float32).at[:13].set(month_table)
    ht32 = jnp.zeros((32, _D), jnp.float32).at[:26].set(hour_table)
    r0 = jnp.concatenate([query_table[0:1], lastv_table[0:1]], axis=0)
    return _tc_head(
        month.astype(i32), hour.astype(i32), qu, lv, mt16, ht32, r0,
        sum_q, sum_lv,
        W1, b1.reshape(1, -1), W2, b2.reshape(1, -1), W3, b3.reshape(1, -1))


# speedup vs baseline: 24.9428x; 1.2776x over previous
import functools

import jax
import jax.numpy as jnp
import numpy as np
from jax import lax
from jax.experimental import pallas as pl
from jax.experimental.pallas import tpu as pltpu
from jax.experimental.pallas import tpu_sc as plsc

_B = 16384
_D = 32
_L = 50
_NW = 32
_ROWS_W = _B // _NW
_CH = 8
_IDXROWS = _CH // 2
_NCHUNK = _ROWS_W // _CH


_IR_W = _ROWS_W // 2
_NBUF = 4


def _sc_pool_sums(q_idx2, lv_idx2, q_table, lv_table):
    mesh = plsc.VectorSubcoreMesh(core_axis_name="c", subcore_axis_name="s")

    @functools.partial(
        pl.kernel,
        mesh=mesh,
        out_type=(
            jax.ShapeDtypeStruct((_B, _D), jnp.float32),
            jax.ShapeDtypeStruct((_B, _D), jnp.float32),
        ),
        scratch_types=[
            pltpu.VMEM((_IR_W, 2 * _L), jnp.int32),
            pltpu.VMEM((_NBUF, 2 * _L, _D), jnp.float32),
            pltpu.VMEM((_ROWS_W, _D), jnp.float32),
            [pltpu.SemaphoreType.DMA] * _NBUF,
        ],
        compiler_params=pltpu.CompilerParams(use_tc_tiling_on_sc=False),
    )
    def k(qi, lvi, qt, lvt, out_q, out_lv, idx_v, gat_v, sums_v, sems):
        cid = lax.axis_index("c")
        sid = lax.axis_index("s")
        wid = sid * 2 + cid

        def reduce_row(slot, o, dst):
            a = [gat_v[slot, o + t, pl.ds(0, 16)] for t in range(4)]
            b = [gat_v[slot, o + t, pl.ds(16, 16)] for t in range(4)]
            for t in range(4, _L):
                a[t % 4] = a[t % 4] + gat_v[slot, o + t, pl.ds(0, 16)]
                b[t % 4] = b[t % 4] + gat_v[slot, o + t, pl.ds(16, 16)]
            sums_v[dst, pl.ds(0, 16)] = (a[0] + a[1]) + (a[2] + a[3])
            sums_v[dst, pl.ds(16, 16)] = (b[0] + b[1]) + (b[2] + b[3])

        for idx_hbm, tab_hbm, out_hbm in ((qi, qt, out_q), (lvi, lvt, out_lv)):
            pltpu.sync_copy(idx_hbm.at[pl.ds(wid * _IR_W, _IR_W)], idx_v)
            for b in range(_NBUF):
                pltpu.async_copy(tab_hbm.at[idx_v.at[b]], gat_v.at[b],
                                 sems[b])

            def body(jj, carry, tab_hbm=tab_hbm):
                for b in range(_NBUF):
                    row = jj * _NBUF + b
                    pltpu.make_async_copy(
                        tab_hbm.at[idx_v.at[row]], gat_v.at[b],
                        sems[b]).wait()
                    reduce_row(b, 0, 2 * row)
                    reduce_row(b, _L, 2 * row + 1)

                    @pl.when(row + _NBUF < _IR_W)
                    def _refire(b=b, row=row):
                        pltpu.async_copy(
                            tab_hbm.at[idx_v.at[row + _NBUF]], gat_v.at[b],
                            sems[b])
                return carry

            lax.fori_loop(0, _IR_W // _NBUF, body, 0)
            pltpu.sync_copy(sums_v, out_hbm.at[pl.ds(wid * _ROWS_W, _ROWS_W)])

    return k(q_idx2, lv_idx2, q_table, lv_table)


def _tc_head(month, hour, query, lastv, mconsts, hconsts, r0,
             sum_q, sum_lv, W1, b1, W2, b2, W3, b3):
    Rm, im, Wm, Wmhi, Wmlo = mconsts
    Rh, ih, Wh, Whhi, Whlo = hconsts
    bm = 2048
    grid = _B // bm
    hp = lax.Precision.HIGHEST
    lo = lax.Precision.DEFAULT
    f32 = jnp.float32

    def body(mo, ho, qu, lv, Rm_, im_, Wm_, Wmhi_, Wmlo_,
             Rh_, ih_, Wh_, Whhi_, Whlo_, r0_, sq, slv,
             W1_, b1_, W2_, b2_, W3_, b3_, out):
        def pooled_small(idx, R, iot, W, Whi, Wlo, npos):
            bf = jnp.bfloat16
            rep = jax.lax.dot(idx.astype(bf), R[...],
                              preferred_element_type=f32)
            oh = (rep == iot[...]).astype(bf)
            Z = (jax.lax.dot(oh, Whi[...], preferred_element_type=f32)
                 + jax.lax.dot(oh, Wlo[...], preferred_element_type=f32))
            s, n0 = Z[:, :_D], Z[:, _D:_D + 1]
            return (s - n0 * W[0:1, :_D]) / jnp.maximum(float(npos) - n0, 1.0)

        m = pooled_small(mo[...], Rm_, im_, Wm_, Wmhi_, Wmlo_, 20)
        h = pooled_small(ho[...], Rh_, ih_, Wh_, Whhi_, Whlo_, 20)

        n0q = jnp.sum((qu[...] == 0).astype(f32), axis=1, keepdims=True)
        q = (sq[...] - n0q * r0_[0:1, :]) / jnp.maximum(50.0 - n0q, 1.0)
        n0l = jnp.sum((lv[...] == 0).astype(f32), axis=1, keepdims=True)
        lvp = (slv[...] - n0l * r0_[1:2, :]) / jnp.maximum(50.0 - n0l, 1.0)

        x = jnp.concatenate([m, h, q, lvp], axis=1)
        x = jnp.maximum(jnp.dot(x, W1_[...], precision=hp) + b1_[...], 0.0)
        x = jnp.maximum(jnp.dot(x, W2_[...], precision=hp) + b2_[...], 0.0)
        out[...] = jnp.dot(x, W3_[...], precision=hp) + b3_[...]

    row_spec = lambda cols: pl.BlockSpec((bm, cols), lambda i: (i, 0))
    full_spec = lambda rows, cols: pl.BlockSpec((rows, cols), lambda i: (0, 0))
    return pl.pallas_call(
        body,
        grid=(grid,),
        in_specs=[
            row_spec(20), row_spec(20), row_spec(_L), row_spec(_L),
            full_spec(*Rm.shape), full_spec(*im.shape), full_spec(*Wm.shape),
            full_spec(*Wmhi.shape), full_spec(*Wmlo.shape),
            full_spec(*Rh.shape), full_spec(*ih.shape), full_spec(*Wh.shape),
            full_spec(*Whhi.shape), full_spec(*Whlo.shape),
            full_spec(2, _D),
            row_spec(_D), row_spec(_D),
            full_spec(128, 128), full_spec(1, 128),
            full_spec(128, 64), full_spec(1, 64),
            full_spec(64, 32), full_spec(1, 32),
        ],
        out_specs=pl.BlockSpec((bm, _D), lambda i: (i, 0)),
        out_shape=jax.ShapeDtypeStruct((_B, _D), jnp.float32),
    )(month, hour, query, lastv, Rm, im, Wm, Wmhi, Wmlo,
      Rh, ih, Wh, Whhi, Whlo, r0, sum_q, sum_lv,
      W1, b1, W2, b2, W3, b3)


def _onehot_consts(table, ncls_pad, npos):
    ncls = table.shape[0]
    K = npos * ncls_pad
    pos = np.arange(K) // ncls_pad
    cls = np.arange(K) % ncls_pad
    R = np.zeros((npos, K), np.float32)
    R[pos, np.arange(K)] = 1.0
    iot = cls[None, :].astype(np.float32)
    tabp = jnp.zeros((ncls_pad, _D + 1), jnp.float32)
    tabp = tabp.at[:ncls, :_D].set(table)
    tabp = tabp.at[0, _D].set(1.0)
    W = tabp[cls, :]
    bf = jnp.bfloat16
    Whi = W.astype(bf)
    Wlo = (W - Whi.astype(jnp.float32)).astype(bf)
    return (jnp.asarray(R, bf), jnp.asarray(iot, jnp.float32), W, Whi, Wlo)


def kernel(month, hour, query, last_viewed, month_table, hour_table,
           query_table, lastv_table, W1, b1, W2, b2, W3, b3):
    i32 = jnp.int32
    qu = query.astype(i32)
    lv = last_viewed.astype(i32)
    sum_q, sum_lv = _sc_pool_sums(
        qu.reshape(_B // 2, 2 * _L), lv.reshape(_B // 2, 2 * _L),
        query_table, lastv_table)
    mconsts = _onehot_consts(month_table, 16, 20)
    hconsts = _onehot_consts(hour_table, 32, 20)
    r0 = jnp.concatenate([query_table[0:1], lastv_table[0:1]], axis=0)
    return _tc_head(
        month.astype(i32), hour.astype(i32), qu, lv,
        mconsts, hconsts, r0, sum_q, sum_lv,
        W1, b1.reshape(1, -1), W2, b2.reshape(1, -1), W3, b3.reshape(1, -1))


# speedup vs baseline: 29.6591x; 1.1891x over previous
import functools

import jax
import jax.numpy as jnp
import numpy as np
from jax import lax
from jax.experimental import pallas as pl
from jax.experimental.pallas import tpu as pltpu
from jax.experimental.pallas import tpu_sc as plsc

_B = 16384
_D = 32
_L = 50
_NW = 32
_ROWS_W = _B // _NW
_CH = 8
_IDXROWS = _CH // 2
_NCHUNK = _ROWS_W // _CH


_IR_W = _ROWS_W // 2
_NBUF = 4


def _sc_pool_sums(q_idx2, lv_idx2, q_table, lv_table):
    mesh = plsc.VectorSubcoreMesh(core_axis_name="c", subcore_axis_name="s")

    @functools.partial(
        pl.kernel,
        mesh=mesh,
        out_type=(
            jax.ShapeDtypeStruct((_B, _D), jnp.float32),
            jax.ShapeDtypeStruct((_B, _D), jnp.float32),
        ),
        scratch_types=[
            pltpu.VMEM((_IR_W, 2 * _L), jnp.int32),
            pltpu.VMEM((_NBUF, 2 * _L, _D), jnp.float32),
            pltpu.VMEM((_ROWS_W, _D), jnp.float32),
            [pltpu.SemaphoreType.DMA] * _NBUF,
        ],
        compiler_params=pltpu.CompilerParams(use_tc_tiling_on_sc=False),
    )
    def k(qi, lvi, qt, lvt, out_q, out_lv, idx_v, gat_v, sums_v, sems):
        cid = lax.axis_index("c")
        sid = lax.axis_index("s")
        wid = sid * 2 + cid

        def reduce_row(slot, o, dst):
            a = [gat_v[slot, o + t, pl.ds(0, 16)] for t in range(4)]
            b = [gat_v[slot, o + t, pl.ds(16, 16)] for t in range(4)]
            for t in range(4, _L):
                a[t % 4] = a[t % 4] + gat_v[slot, o + t, pl.ds(0, 16)]
                b[t % 4] = b[t % 4] + gat_v[slot, o + t, pl.ds(16, 16)]
            sums_v[dst, pl.ds(0, 16)] = (a[0] + a[1]) + (a[2] + a[3])
            sums_v[dst, pl.ds(16, 16)] = (b[0] + b[1]) + (b[2] + b[3])

        for idx_hbm, tab_hbm, out_hbm in ((qi, qt, out_q), (lvi, lvt, out_lv)):
            pltpu.sync_copy(idx_hbm.at[pl.ds(wid * _IR_W, _IR_W)], idx_v)
            for b in range(_NBUF):
                pltpu.async_copy(tab_hbm.at[idx_v.at[b]], gat_v.at[b],
                                 sems[b])

            def body(jj, carry, tab_hbm=tab_hbm):
                for b in range(_NBUF):
                    row = jj * _NBUF + b
                    pltpu.make_async_copy(
                        tab_hbm.at[idx_v.at[row]], gat_v.at[b],
                        sems[b]).wait()
                    reduce_row(b, 0, 2 * row)
                    reduce_row(b, _L, 2 * row + 1)

                    @pl.when(row + _NBUF < _IR_W)
                    def _refire(b=b, row=row):
                        pltpu.async_copy(
                            tab_hbm.at[idx_v.at[row + _NBUF]], gat_v.at[b],
                            sems[b])
                return carry

            lax.fori_loop(0, _IR_W // _NBUF, body, 0)
            pltpu.sync_copy(sums_v, out_hbm.at[pl.ds(wid * _ROWS_W, _ROWS_W)])

    return k(q_idx2, lv_idx2, q_table, lv_table)


_BM = 2048
_HP = lax.Precision.HIGHEST


def _row_spec(cols):
    return pl.BlockSpec((_BM, cols), lambda i: (i, 0))


def _full_spec(rows, cols):
    return pl.BlockSpec((rows, cols), lambda i: (0, 0))


def _tc_head_a(month, hour, query, lastv, mconsts, hconsts, W1ab, b1):
    Rm, im, Wm, Wmhi, Wmlo = mconsts
    Rh, ih, Wh, Whhi, Whlo = hconsts
    f32 = jnp.float32

    def body(mo, ho, qu, lv, Rm_, im_, Wm_, Wmhi_, Wmlo_,
             Rh_, ih_, Wh_, Whhi_, Whlo_, W1ab_, b1_, ypart, aux):
        def pooled_small(idx, R, iot, W, Whi, Wlo, npos):
            bf = jnp.bfloat16
            rep = jax.lax.dot(idx.astype(bf), R[...],
                              preferred_element_type=f32)
            oh = (rep == iot[...]).astype(bf)
            Z = (jax.lax.dot(oh, Whi[...], preferred_element_type=f32)
                 + jax.lax.dot(oh, Wlo[...], preferred_element_type=f32))
            s, n0 = Z[:, :_D], Z[:, _D:_D + 1]
            return (s - n0 * W[0:1, :_D]) / jnp.maximum(float(npos) - n0, 1.0)

        m = pooled_small(mo[...], Rm_, im_, Wm_, Wmhi_, Wmlo_, 20)
        h = pooled_small(ho[...], Rh_, ih_, Wh_, Whhi_, Whlo_, 20)
        mh = jnp.concatenate([m, h], axis=1)
        ypart[...] = jnp.dot(mh, W1ab_[...], precision=_HP) + b1_[...]

        n0q = jnp.sum((qu[...] == 0).astype(f32), axis=1, keepdims=True)
        n0l = jnp.sum((lv[...] == 0).astype(f32), axis=1, keepdims=True)
        aux[...] = jnp.concatenate(
            [n0q, n0l, jnp.zeros((_BM, 6), f32)], axis=1)

    return pl.pallas_call(
        body,
        grid=(_B // _BM,),
        in_specs=[
            _row_spec(20), _row_spec(20), _row_spec(_L), _row_spec(_L),
            _full_spec(*Rm.shape), _full_spec(*im.shape),
            _full_spec(*Wm.shape),
            _full_spec(*Wmhi.shape), _full_spec(*Wmlo.shape),
            _full_spec(*Rh.shape), _full_spec(*ih.shape),
            _full_spec(*Wh.shape),
            _full_spec(*Whhi.shape), _full_spec(*Whlo.shape),
            _full_spec(64, 128), _full_spec(1, 128),
        ],
        out_specs=[_row_spec(128), _row_spec(8)],
        out_shape=[jax.ShapeDtypeStruct((_B, 128), jnp.float32),
                   jax.ShapeDtypeStruct((_B, 8), jnp.float32)],
    )(month, hour, query, lastv, Rm, im, Wm, Wmhi, Wmlo,
      Rh, ih, Wh, Whhi, Whlo, W1ab, b1)


def _tc_head_b(ypart, aux, sum_q, sum_lv, r0, W1c, W1d, W2, b2, W3, b3):
    f32 = jnp.float32

    def body(yp, ax, sq, slv, r0_, W1c_, W1d_, W2_, b2_, W3_, b3_, out):
        n0q = ax[:, 0:1]
        n0l = ax[:, 1:2]
        rcq = 1.0 / jnp.maximum(50.0 - n0q, 1.0)
        rcl = 1.0 / jnp.maximum(50.0 - n0l, 1.0)
        q0W = jnp.dot(r0_[0:1, :], W1c_[...], precision=_HP)
        l0W = jnp.dot(r0_[1:2, :], W1d_[...], precision=_HP)
        yq = (jnp.dot(sq[...], W1c_[...], precision=_HP) - n0q * q0W) * rcq
        yl = (jnp.dot(slv[...], W1d_[...], precision=_HP) - n0l * l0W) * rcl
        x = jnp.maximum(yp[...] + yq + yl, 0.0)
        x = jnp.maximum(jnp.dot(x, W2_[...], precision=_HP) + b2_[...], 0.0)
        out[...] = jnp.dot(x, W3_[...], precision=_HP) + b3_[...]

    return pl.pallas_call(
        body,
        grid=(_B // _BM,),
        in_specs=[
            _row_spec(128), _row_spec(8), _row_spec(_D), _row_spec(_D),
            _full_spec(2, _D),
            _full_spec(_D, 128), _full_spec(_D, 128),
            _full_spec(128, 64), _full_spec(1, 64),
            _full_spec(64, 32), _full_spec(1, 32),
        ],
        out_specs=_row_spec(_D),
        out_shape=jax.ShapeDtypeStruct((_B, _D), jnp.float32),
    )(ypart, aux, sum_q, sum_lv, r0, W1c, W1d, W2, b2, W3, b3)


def _onehot_consts(table, ncls_pad, npos):
    ncls = table.shape[0]
    K = npos * ncls_pad
    pos = np.arange(K) // ncls_pad
    cls = np.arange(K) % ncls_pad
    R = np.zeros((npos, K), np.float32)
    R[pos, np.arange(K)] = 1.0
    iot = cls[None, :].astype(np.float32)
    tabp = jnp.zeros((ncls_pad, _D + 1), jnp.float32)
    tabp = tabp.at[:ncls, :_D].set(table)
    tabp = tabp.at[0, _D].set(1.0)
    W = tabp[cls, :]
    bf = jnp.bfloat16
    Whi = W.astype(bf)
    Wlo = (W - Whi.astype(jnp.float32)).astype(bf)
    return (jnp.asarray(R, bf), jnp.asarray(iot, jnp.float32), W, Whi, Wlo)


def kernel(month, hour, query, last_viewed, month_table, hour_table,
           query_table, lastv_table, W1, b1, W2, b2, W3, b3):
    i32 = jnp.int32
    qu = query.astype(i32)
    lv = last_viewed.astype(i32)
    mconsts = _onehot_consts(month_table, 16, 20)
    hconsts = _onehot_consts(hour_table, 32, 20)
    r0 = jnp.concatenate([query_table[0:1], lastv_table[0:1]], axis=0)
    ypart, aux = _tc_head_a(
        month.astype(i32), hour.astype(i32), qu, lv, mconsts, hconsts,
        W1[0:64], b1.reshape(1, -1))
    sum_q, sum_lv = _sc_pool_sums(
        qu.reshape(_B // 2, 2 * _L), lv.reshape(_B // 2, 2 * _L),
        query_table, lastv_table)
    return _tc_head_b(
        ypart, aux, sum_q, sum_lv, r0, W1[64:96], W1[96:128],
        W2, b2.reshape(1, -1), W3, b3.reshape(1, -1))


# speedup vs baseline: 33.9847x; 1.1458x over previous
import functools

import jax
import jax.numpy as jnp
import numpy as np
from jax import lax
from jax.experimental import pallas as pl
from jax.experimental.pallas import tpu as pltpu
from jax.experimental.pallas import tpu_sc as plsc

_B = 16384
_D = 32
_L = 50
_NW = 32
_ROWS_W = _B // _NW
_CH = 8
_IDXROWS = _CH // 2
_NCHUNK = _ROWS_W // _CH


_NBUF = 8


def _sc_pool_sum(idx, table):
    mesh = plsc.VectorSubcoreMesh(core_axis_name="c", subcore_axis_name="s")

    @functools.partial(
        pl.kernel,
        mesh=mesh,
        out_type=jax.ShapeDtypeStruct((_B, _D), jnp.float32),
        scratch_types=[
            pltpu.VMEM((_ROWS_W, _L), jnp.int32),
            pltpu.VMEM((_NBUF, _L, _D), jnp.float32),
            pltpu.VMEM((_ROWS_W, _D), jnp.float32),
            [pltpu.SemaphoreType.DMA] * _NBUF,
        ],
        compiler_params=pltpu.CompilerParams(use_tc_tiling_on_sc=False),
    )
    def k(idx_hbm, tab_hbm, out_hbm, idx_v, gat_v, sums_v, sems):
        cid = lax.axis_index("c")
        sid = lax.axis_index("s")
        wid = sid * 2 + cid

        def reduce_row(slot, dst):
            a = [gat_v[slot, t, pl.ds(0, 16)] for t in range(4)]
            b = [gat_v[slot, t, pl.ds(16, 16)] for t in range(4)]
            for t in range(4, _L):
                a[t % 4] = a[t % 4] + gat_v[slot, t, pl.ds(0, 16)]
                b[t % 4] = b[t % 4] + gat_v[slot, t, pl.ds(16, 16)]
            sums_v[dst, pl.ds(0, 16)] = (a[0] + a[1]) + (a[2] + a[3])
            sums_v[dst, pl.ds(16, 16)] = (b[0] + b[1]) + (b[2] + b[3])

        pltpu.sync_copy(idx_hbm.at[pl.ds(wid * _ROWS_W, _ROWS_W)], idx_v)
        for b in range(_NBUF):
            pltpu.async_copy(tab_hbm.at[idx_v.at[b]], gat_v.at[b], sems[b])

        def body(jj, carry):
            for b in range(_NBUF):
                row = jj * _NBUF + b
                pltpu.make_async_copy(
                    tab_hbm.at[idx_v.at[row]], gat_v.at[b], sems[b]).wait()
                reduce_row(b, row)

                @pl.when(row + _NBUF < _ROWS_W)
                def _refire(b=b, row=row):
                    pltpu.async_copy(
                        tab_hbm.at[idx_v.at[row + _NBUF]], gat_v.at[b],
                        sems[b])
            return carry

        lax.fori_loop(0, _ROWS_W // _NBUF, body, 0)
        pltpu.sync_copy(sums_v, out_hbm.at[pl.ds(wid * _ROWS_W, _ROWS_W)])

    return k(idx, table)


_BM = 2048
_HP = lax.Precision.HIGHEST


def _row_spec(cols):
    return pl.BlockSpec((_BM, cols), lambda i: (i, 0))


def _full_spec(rows, cols):
    return pl.BlockSpec((rows, cols), lambda i: (0, 0))


def _tc_head_a(month, hour, query, lastv, mconsts, hconsts, W1ab, b1):
    Rm, im, Wm, Wmhi, Wmlo = mconsts
    Rh, ih, Wh, Whhi, Whlo = hconsts
    f32 = jnp.float32

    def body(mo, ho, qu, lv, Rm_, im_, Wm_, Wmhi_, Wmlo_,
             Rh_, ih_, Wh_, Whhi_, Whlo_, W1ab_, b1_, ypart, aux):
        def pooled_small(idx, R, iot, W, Whi, Wlo, npos):
            bf = jnp.bfloat16
            rep = jax.lax.dot(idx.astype(bf), R[...],
                              preferred_element_type=f32)
            oh = (rep == iot[...]).astype(bf)
            Z = (jax.lax.dot(oh, Whi[...], preferred_element_type=f32)
                 + jax.lax.dot(oh, Wlo[...], preferred_element_type=f32))
            s, n0 = Z[:, :_D], Z[:, _D:_D + 1]
            return (s - n0 * W[0:1, :_D]) / jnp.maximum(float(npos) - n0, 1.0)

        m = pooled_small(mo[...], Rm_, im_, Wm_, Wmhi_, Wmlo_, 20)
        h = pooled_small(ho[...], Rh_, ih_, Wh_, Whhi_, Whlo_, 20)
        mh = jnp.concatenate([m, h], axis=1)
        ypart[...] = jnp.dot(mh, W1ab_[...], precision=_HP) + b1_[...]

        n0q = jnp.sum((qu[...] == 0).astype(f32), axis=1, keepdims=True)
        n0l = jnp.sum((lv[...] == 0).astype(f32), axis=1, keepdims=True)
        aux[...] = jnp.concatenate(
            [n0q, n0l, jnp.zeros((_BM, 6), f32)], axis=1)

    return pl.pallas_call(
        body,
        grid=(_B // _BM,),
        in_specs=[
            _row_spec(20), _row_spec(20), _row_spec(_L), _row_spec(_L),
            _full_spec(*Rm.shape), _full_spec(*im.shape),
            _full_spec(*Wm.shape),
            _full_spec(*Wmhi.shape), _full_spec(*Wmlo.shape),
            _full_spec(*Rh.shape), _full_spec(*ih.shape),
            _full_spec(*Wh.shape),
            _full_spec(*Whhi.shape), _full_spec(*Whlo.shape),
            _full_spec(64, 128), _full_spec(1, 128),
        ],
        out_specs=[_row_spec(128), _row_spec(8)],
        out_shape=[jax.ShapeDtypeStruct((_B, 128), jnp.float32),
                   jax.ShapeDtypeStruct((_B, 8), jnp.float32)],
    )(month, hour, query, lastv, Rm, im, Wm, Wmhi, Wmlo,
      Rh, ih, Wh, Whhi, Whlo, W1ab, b1)


def _tc_head_b(ypart, aux, sum_q, sum_lv, r0, W1c, W1d, W2, b2, W3, b3):
    f32 = jnp.float32

    def body(yp, ax, sq, slv, r0_, W1c_, W1d_, W2_, b2_, W3_, b3_, out):
        n0q = ax[:, 0:1]
        n0l = ax[:, 1:2]
        rcq = 1.0 / jnp.maximum(50.0 - n0q, 1.0)
        rcl = 1.0 / jnp.maximum(50.0 - n0l, 1.0)
        q0W = jnp.dot(r0_[0:1, :], W1c_[...], precision=_HP)
        l0W = jnp.dot(r0_[1:2, :], W1d_[...], precision=_HP)
        yq = (jnp.dot(sq[...], W1c_[...], precision=_HP) - n0q * q0W) * rcq
        yl = (jnp.dot(slv[...], W1d_[...], precision=_HP) - n0l * l0W) * rcl
        x = jnp.maximum(yp[...] + yq + yl, 0.0)
        x = jnp.maximum(jnp.dot(x, W2_[...], precision=_HP) + b2_[...], 0.0)
        out[...] = jnp.dot(x, W3_[...], precision=_HP) + b3_[...]

    return pl.pallas_call(
        body,
        grid=(_B // _BM,),
        in_specs=[
            _row_spec(128), _row_spec(8), _row_spec(_D), _row_spec(_D),
            _full_spec(2, _D),
            _full_spec(_D, 128), _full_spec(_D, 128),
            _full_spec(128, 64), _full_spec(1, 64),
            _full_spec(64, 32), _full_spec(1, 32),
        ],
        out_specs=_row_spec(_D),
        out_shape=jax.ShapeDtypeStruct((_B, _D), jnp.float32),
    )(ypart, aux, sum_q, sum_lv, r0, W1c, W1d, W2, b2, W3, b3)


def _onehot_consts(table, ncls_pad, npos):
    ncls = table.shape[0]
    K = npos * ncls_pad
    pos = np.arange(K) // ncls_pad
    cls = np.arange(K) % ncls_pad
    R = np.zeros((npos, K), np.float32)
    R[pos, np.arange(K)] = 1.0
    iot = cls[None, :].astype(np.float32)
    tabp = jnp.zeros((ncls_pad, _D + 1), jnp.float32)
    tabp = tabp.at[:ncls, :_D].set(table)
    tabp = tabp.at[0, _D].set(1.0)
    W = tabp[cls, :]
    bf = jnp.bfloat16
    Whi = W.astype(bf)
    Wlo = (W - Whi.astype(jnp.float32)).astype(bf)
    return (jnp.asarray(R, bf), jnp.asarray(iot, jnp.float32), W, Whi, Wlo)


def kernel(month, hour, query, last_viewed, month_table, hour_table,
           query_table, lastv_table, W1, b1, W2, b2, W3, b3):
    i32 = jnp.int32
    qu = query.astype(i32)
    lv = last_viewed.astype(i32)
    mconsts = _onehot_consts(month_table, 16, 20)
    hconsts = _onehot_consts(hour_table, 32, 20)
    r0 = jnp.concatenate([query_table[0:1], lastv_table[0:1]], axis=0)
    ypart, aux = _tc_head_a(
        month.astype(i32), hour.astype(i32), qu, lv, mconsts, hconsts,
        W1[0:64], b1.reshape(1, -1))
    sum_q = _sc_pool_sum(qu, query_table)
    sum_lv = _sc_pool_sum(lv, lastv_table)
    return _tc_head_b(
        ypart, aux, sum_q, sum_lv, r0, W1[64:96], W1[96:128],
        W2, b2.reshape(1, -1), W3, b3.reshape(1, -1))
